# Initial kernel scaffold; baseline (speedup 1.0000x reference)
import functools

import jax
import jax.numpy as jnp
from jax import lax
from jax.experimental import pallas as pl
from jax.experimental.pallas import tpu as pltpu
from jax.experimental.pallas import tpu_sc as plsc

N = 10000
E = 320000
D = 128
G = 128

NC = 2
NS = 16
NW = NC * NS

CH = 128
ET = 10240
EP = ET * NW
NCH = ET // CH
IDXROWS = EP // CH

NA = 10240
RPT = NA // NS
DW = 16

NPT = NA // NW
RCH = 80
NRCH = NPT // RCH

BLK = 1000
NBLK = N // BLK

_MESH = plsc.VectorSubcoreMesh(core_axis_name="c", subcore_axis_name="s")


@functools.partial(
    pl.kernel,
    out_type=(
        jax.ShapeDtypeStruct((NC, NA, DW), jnp.float32),
        jax.ShapeDtypeStruct((NA, D), jnp.float32),
    ),
    mesh=_MESH,
    scratch_types=[
        pltpu.VMEM((NCH, CH), jnp.int32),
        pltpu.VMEM((CH, DW), jnp.float32),
        pltpu.VMEM((G,), jnp.int32),
        pltpu.VMEM((RCH,), jnp.int32),
        pltpu.VMEM((RCH,), jnp.int32),
        pltpu.VMEM((RCH, D), jnp.float32),
        pltpu.VMEM_SHARED((NA, DW), jnp.float32),
        pltpu.SemaphoreType.DMA,
    ],
)
def _sc_stage_a(dst2d, batch_p, root_idx, x, ones_rows, zdeg,
                deg_out, rbx_out,
                dstv, onesv, rootv, bchv, idxv, rowsv, dacc, sem):
    cid = lax.axis_index("c")
    sid = lax.axis_index("s")
    wid = cid * NS + sid

    row0 = pl.multiple_of(sid * RPT, 8)
    pltpu.sync_copy(zdeg, dacc.at[pl.ds(row0, RPT)])
    pltpu.sync_copy(ones_rows, onesv)
    pltpu.sync_copy(dst2d.at[pl.ds(pl.multiple_of(wid * NCH, 8), NCH)], dstv)
    plsc.subcore_barrier()

    def deg_body(j, carry):
        pltpu.sync_copy(onesv, dacc.at[dstv.at[j]], add=True)
        return carry
    lax.fori_loop(0, NCH, deg_body, 0)

    pltpu.sync_copy(root_idx, rootv)
    for r in range(NRCH):
        off = pl.multiple_of(wid * NPT + r * RCH, 8)
        pltpu.sync_copy(batch_p.at[pl.ds(off, RCH)], bchv)
        for gblk in range(RCH // 16):
            bvals = bchv[pl.ds(gblk * 16, 16)]
            idxv[pl.ds(gblk * 16, 16)] = plsc.load_gather(rootv, [bvals])
        pltpu.async_copy(x.at[idxv], rowsv, sem).wait()
        pltpu.sync_copy(rowsv, rbx_out.at[pl.ds(off, RCH)])

    plsc.subcore_barrier()
    pltpu.sync_copy(dacc.at[pl.ds(row0, RPT)], deg_out.at[cid, pl.ds(row0, RPT)])


def _make_agg(with_root):
    outs = [jax.ShapeDtypeStruct((NC, NA, D), jnp.float32)]
    scratch = [
        pltpu.VMEM((NCH, CH), jnp.int32),
        pltpu.VMEM((NCH, CH), jnp.int32),
        pltpu.VMEM((CH, D), jnp.float32),
        pltpu.VMEM_SHARED((NA, D), jnp.float32),
        pltpu.SemaphoreType.DMA,
    ]
    if with_root:
        outs.append(jax.ShapeDtypeStruct((G, D), jnp.float32))
        scratch += [
            pltpu.VMEM((G,), jnp.int32),
            pltpu.VMEM((G, D), jnp.float32),
            pltpu.SemaphoreType.DMA,
        ]

    @functools.partial(
        pl.kernel,
        out_type=tuple(outs) if with_root else outs[0],
        mesh=_MESH,
        scratch_types=scratch,
    )
    def _agg(*refs):
        if with_root:
            (src2d, dst2d, table, zrows, root_idx, xroot_src,
             acc_out, root_out,
             srcv, dstv, rowsv, acc, sem, rootv, rrowsv, rsem) = refs
        else:
            (src2d, dst2d, table, zrows,
             acc_out,
             srcv, dstv, rowsv, acc, sem) = refs
        cid = lax.axis_index("c")
        sid = lax.axis_index("s")
        wid = cid * NS + sid

        row0 = pl.multiple_of(sid * RPT, 8)
        pltpu.sync_copy(zrows, acc.at[pl.ds(row0, RPT)])

        idx0 = pl.multiple_of(wid * NCH, 8)
        pltpu.sync_copy(src2d.at[pl.ds(idx0, NCH)], srcv)
        pltpu.sync_copy(dst2d.at[pl.ds(idx0, NCH)], dstv)
        plsc.subcore_barrier()

        def body(j, carry):
            pltpu.async_copy(table.at[srcv.at[j]], rowsv, sem).wait()
            pltpu.sync_copy(rowsv, acc.at[dstv.at[j]], add=True)
            return carry
        lax.fori_loop(0, NCH, body, 0)

        if with_root:
            @pl.when(jnp.logical_and(cid == 0, sid == 0))
            def _():
                pltpu.sync_copy(root_idx, rootv)
                pltpu.async_copy(xroot_src.at[rootv], rrowsv, rsem).wait()
                pltpu.sync_copy(rrowsv, root_out)

        plsc.subcore_barrier()
        pltpu.sync_copy(acc.at[pl.ds(row0, RPT)],
                        acc_out.at[cid, pl.ds(row0, RPT)])

    return _agg


_sc_agg = _make_agg(False)
_sc_agg_root = _make_agg(True)


def _tc_b_body(x_ref, w1_ref, deg_ref, hs1_ref, dinv_ref):
    deg = deg_ref[0, :, 0:1] + deg_ref[1, :, 0:1] + 1.0
    dinv = lax.rsqrt(deg)
    h1 = jnp.dot(x_ref[...], w1_ref[...], preferred_element_type=jnp.float32)
    hs1_ref[...] = h1 * dinv
    dinv_ref[...] = dinv


def _tc_stage_b(x, W1, deg):
    return pl.pallas_call(
        _tc_b_body,
        grid=(NBLK,),
        in_specs=[
            pl.BlockSpec((BLK, D), lambda i: (i, 0)),
            pl.BlockSpec((D, D), lambda i: (0, 0)),
            pl.BlockSpec((NC, BLK, DW), lambda i: (0, i, 0)),
        ],
        out_specs=[
            pl.BlockSpec((BLK, D), lambda i: (i, 0)),
            pl.BlockSpec((BLK, 1), lambda i: (i, 0)),
        ],
        out_shape=[
            jax.ShapeDtypeStruct((N, D), jnp.float32),
            jax.ShapeDtypeStruct((N, 1), jnp.float32),
        ],
    )(x, W1, deg)


def _tc_d_body(acc_ref, hs1_ref, dinv_ref, rbx_ref, w2a_ref, w2b_ref, b1_ref,
               x2_ref, hs2_ref):
    dinv = dinv_ref[...]
    a = acc_ref[0] + acc_ref[1] + hs1_ref[...]
    x2 = a * dinv + b1_ref[...]
    g = (jnp.dot(jnp.maximum(x2, 0.0), w2a_ref[...],
                 preferred_element_type=jnp.float32)
         + jnp.dot(jnp.maximum(rbx_ref[...], 0.0), w2b_ref[...],
                   preferred_element_type=jnp.float32))
    x2_ref[...] = x2
    hs2_ref[...] = g * dinv


def _tc_stage_d(acc1, hs1, dinv, rbx, W2a, W2b, b1):
    return pl.pallas_call(
        _tc_d_body,
        grid=(NBLK,),
        in_specs=[
            pl.BlockSpec((NC, BLK, D), lambda i: (0, i, 0)),
            pl.BlockSpec((BLK, D), lambda i: (i, 0)),
            pl.BlockSpec((BLK, 1), lambda i: (i, 0)),
            pl.BlockSpec((BLK, D), lambda i: (i, 0)),
            pl.BlockSpec((D, D), lambda i: (0, 0)),
            pl.BlockSpec((D, D), lambda i: (0, 0)),
            pl.BlockSpec((1, D), lambda i: (0, 0)),
        ],
        out_specs=[
            pl.BlockSpec((BLK, D), lambda i: (i, 0)),
            pl.BlockSpec((BLK, D), lambda i: (i, 0)),
        ],
        out_shape=[
            jax.ShapeDtypeStruct((N, D), jnp.float32),
            jax.ShapeDtypeStruct((N, D), jnp.float32),
        ],
    )(acc1, hs1, dinv, rbx, W2a, W2b, b1)


def _tc_f_body(acc2_ref, hs2_ref, dinv_ref, b2_ref, batch_ref, x2root_ref,
               out_ref, sums, cnt):
    i = pl.program_id(0)
    out2 = ((acc2_ref[0] + acc2_ref[1] + hs2_ref[...]) * dinv_ref[...]
            + b2_ref[...])
    f = jnp.maximum(out2, 0.0)
    brow = batch_ref[...]
    oh_t = (lax.broadcasted_iota(jnp.int32, (G, BLK), 0) == brow
            ).astype(jnp.float32)
    psum = jnp.dot(oh_t, f, preferred_element_type=jnp.float32)
    pcnt = jnp.dot(oh_t, jnp.ones((BLK, 1), jnp.float32),
                   preferred_element_type=jnp.float32)

    @pl.when(i == 0)
    def _():
        sums[...] = jnp.zeros_like(sums)
        cnt[...] = jnp.zeros_like(cnt)

    sums[...] += psum
    cnt[...] += pcnt

    @pl.when(i == pl.num_programs(0) - 1)
    def _():
        c = cnt[...]
        out_ref[:, :D] = sums[...] / jnp.maximum(c, 1.0)
        out_ref[:, D:] = jnp.where(c > 0.0, x2root_ref[...], 0.0)


def _tc_stage_f(acc2, hs2, dinv, b2, batch2d, x2root):
    return pl.pallas_call(
        _tc_f_body,
        grid=(NBLK,),
        in_specs=[
            pl.BlockSpec((NC, BLK, D), lambda i: (0, i, 0)),
            pl.BlockSpec((BLK, D), lambda i: (i, 0)),
            pl.BlockSpec((BLK, 1), lambda i: (i, 0)),
            pl.BlockSpec((1, D), lambda i: (0, 0)),
            pl.BlockSpec((1, BLK), lambda i: (i, 0)),
            pl.BlockSpec((G, D), lambda i: (0, 0)),
        ],
        out_specs=pl.BlockSpec((G, 2 * D), lambda i: (0, 0)),
        out_shape=jax.ShapeDtypeStruct((G, 2 * D), jnp.float32),
        scratch_shapes=[
            pltpu.VMEM((G, D), jnp.float32),
            pltpu.VMEM((G, 1), jnp.float32),
        ],
    )(acc2, hs2, dinv, b2, batch2d, x2root)


def kernel(x, bu_edge_index, batch, root_index, W1, b1, W2, b2):
    x = x.astype(jnp.float32)
    ei = bu_edge_index.astype(jnp.int32)
    batch_i = batch.astype(jnp.int32)
    root_i = root_index.astype(jnp.int32)

    src = ei[0]
    dst = ei[1]
    npad = EP - E
    src_p = jnp.concatenate([src, jnp.zeros((npad,), jnp.int32)])
    trash = N + (jnp.arange(npad, dtype=jnp.int32) % (NA - N))
    dst_p = jnp.concatenate([dst, trash])
    src2d = src_p.reshape(IDXROWS, CH)
    dst2d = dst_p.reshape(IDXROWS, CH)
    batch_p = jnp.concatenate([batch_i, jnp.zeros((NA - N,), jnp.int32)])

    ones_rows = jnp.ones((CH, DW), jnp.float32)
    zdeg = jnp.zeros((RPT, DW), jnp.float32)
    zrows = jnp.zeros((RPT, D), jnp.float32)

    deg, rbx_p = _sc_stage_a(dst2d, batch_p, root_i, x, ones_rows, zdeg)

    hs1, dinv = _tc_stage_b(x, W1, deg)

    acc1 = _sc_agg(src2d, dst2d, hs1, zrows)

    W2a = W2[:D]
    W2b = W2[D:]
    x2, hs2 = _tc_stage_d(acc1, hs1, dinv, rbx_p, W2a, W2b,
                          b1.reshape(1, D))

    acc2, x2root = _sc_agg_root(src2d, dst2d, hs2, zrows, root_i, x2)

    out = _tc_stage_f(acc2, hs2, dinv, b2.reshape(1, D),
                      batch_i.reshape(NBLK, BLK), x2root)
    return out


# speedup vs baseline: 9.3345x; 9.3345x over previous
import functools

import jax
import jax.numpy as jnp
from jax import lax
from jax.experimental import pallas as pl
from jax.experimental.pallas import tpu as pltpu
from jax.experimental.pallas import tpu_sc as plsc

N = 10000
E = 320000
D = 128
G = 128

NC = 2
NS = 16
NW = NC * NS

CH = 128
ET = 10240
EP = ET * NW
NCH = ET // CH
IDXROWS = EP // CH

NA = 10240
RPT = NA // NS
DW = 16

NPT = NA // NW
RCH = 80
NRCH = NPT // RCH

BLK = 1000
NBLK = N // BLK

_MESH = plsc.VectorSubcoreMesh(core_axis_name="c", subcore_axis_name="s")


@functools.partial(
    pl.kernel,
    out_type=(
        jax.ShapeDtypeStruct((NC, NA), jnp.float32),
        jax.ShapeDtypeStruct((NA, D), jnp.float32),
    ),
    mesh=_MESH,
    scratch_types=[
        pltpu.VMEM((NCH, CH), jnp.int32),
        pltpu.VMEM((CH,), jnp.float32),
        pltpu.VMEM((G,), jnp.int32),
        pltpu.VMEM((RCH,), jnp.int32),
        pltpu.VMEM((RCH,), jnp.int32),
        pltpu.VMEM((RCH, D), jnp.float32),
        pltpu.VMEM_SHARED((NA,), jnp.float32),
        pltpu.SemaphoreType.DMA,
    ],
    compiler_params=pltpu.CompilerParams(needs_layout_passes=False),
)
def _sc_stage_a(dst2d, batch_p, root_idx, x, ones_rows, zdeg,
                deg_out, rbx_out,
                dstv, onesv, rootv, bchv, idxv, rowsv, dacc, sem):
    cid = lax.axis_index("c")
    sid = lax.axis_index("s")
    wid = cid * NS + sid

    row0 = pl.multiple_of(sid * RPT, 8)
    pltpu.sync_copy(zdeg, dacc.at[pl.ds(row0, RPT)])
    pltpu.sync_copy(ones_rows, onesv)
    pltpu.sync_copy(dst2d.at[pl.ds(pl.multiple_of(wid * NCH, 8), NCH)], dstv)
    plsc.subcore_barrier()

    def deg_body(j, carry):
        pltpu.sync_copy(onesv, dacc.at[dstv.at[j]], add=True)
        return carry
    lax.fori_loop(0, NCH, deg_body, 0)

    pltpu.sync_copy(root_idx, rootv)
    for r in range(NRCH):
        off = pl.multiple_of(wid * NPT + r * RCH, 8)
        pltpu.sync_copy(batch_p.at[pl.ds(off, RCH)], bchv)
        for gblk in range(RCH // 16):
            bvals = bchv[pl.ds(gblk * 16, 16)]
            idxv[pl.ds(gblk * 16, 16)] = plsc.load_gather(rootv, [bvals])
        pltpu.async_copy(x.at[idxv], rowsv, sem).wait()
        pltpu.sync_copy(rowsv, rbx_out.at[pl.ds(off, RCH)])

    plsc.subcore_barrier()
    pltpu.sync_copy(dacc.at[pl.ds(row0, RPT)], deg_out.at[cid, pl.ds(row0, RPT)])


def _make_agg(with_root):
    outs = [jax.ShapeDtypeStruct((NC, NA, D), jnp.float32)]
    scratch = [
        pltpu.VMEM((NCH, CH), jnp.int32),
        pltpu.VMEM((NCH, CH), jnp.int32),
        pltpu.VMEM((CH, D), jnp.float32),
        pltpu.VMEM_SHARED((NA, D), jnp.float32),
        pltpu.SemaphoreType.DMA,
    ]
    if with_root:
        outs.append(jax.ShapeDtypeStruct((G, D), jnp.float32))
        scratch += [
            pltpu.VMEM((G,), jnp.int32),
        ]

    @functools.partial(
        pl.kernel,
        out_type=tuple(outs) if with_root else outs[0],
        mesh=_MESH,
        scratch_types=scratch,
    )
    def _agg(*refs):
        if with_root:
            (src2d, dst2d, table, zrows, root_idx, xroot_src,
             acc_out, root_out,
             srcv, dstv, rowsv, acc, sem, rootv) = refs
        else:
            (src2d, dst2d, table, zrows,
             acc_out,
             srcv, dstv, rowsv, acc, sem) = refs
        cid = lax.axis_index("c")
        sid = lax.axis_index("s")
        wid = cid * NS + sid

        row0 = pl.multiple_of(sid * RPT, 8)
        pltpu.sync_copy(zrows, acc.at[pl.ds(row0, RPT)])

        idx0 = pl.multiple_of(wid * NCH, 8)
        pltpu.sync_copy(src2d.at[pl.ds(idx0, NCH)], srcv)
        pltpu.sync_copy(dst2d.at[pl.ds(idx0, NCH)], dstv)
        plsc.subcore_barrier()

        def body(j, carry):
            pltpu.async_copy(table.at[srcv.at[j]], rowsv, sem).wait()
            pltpu.sync_copy(rowsv, acc.at[dstv.at[j]], add=True)
            return carry
        lax.fori_loop(0, NCH, body, 0)

        if with_root:
            @pl.when(jnp.logical_and(cid == 0, sid == 0))
            def _():
                pltpu.sync_copy(root_idx, rootv)
                pltpu.async_copy(xroot_src.at[rootv], rowsv, sem).wait()
                pltpu.sync_copy(rowsv, root_out)

        plsc.subcore_barrier()
        pltpu.sync_copy(acc.at[pl.ds(row0, RPT)],
                        acc_out.at[cid, pl.ds(row0, RPT)])

    return _agg


_sc_agg = _make_agg(False)
_sc_agg_root = _make_agg(True)


def _tc_b_body(x_ref, w1_ref, deg_ref, hs1_ref, dinv_ref):
    deg = deg_ref[:, 0:1] + deg_ref[:, 1:2] + 1.0
    dinv = lax.rsqrt(deg)
    h1 = jnp.dot(x_ref[...], w1_ref[...], preferred_element_type=jnp.float32)
    hs1_ref[...] = h1 * dinv
    dinv_ref[...] = dinv


def _tc_stage_b(x, W1, deg):
    return pl.pallas_call(
        _tc_b_body,
        grid=(NBLK,),
        in_specs=[
            pl.BlockSpec((BLK, D), lambda i: (i, 0)),
            pl.BlockSpec((D, D), lambda i: (0, 0)),
            pl.BlockSpec((BLK, NC), lambda i: (i, 0)),
        ],
        out_specs=[
            pl.BlockSpec((BLK, D), lambda i: (i, 0)),
            pl.BlockSpec((BLK, 1), lambda i: (i, 0)),
        ],
        out_shape=[
            jax.ShapeDtypeStruct((N, D), jnp.float32),
            jax.ShapeDtypeStruct((N, 1), jnp.float32),
        ],
    )(x, W1, deg)


def _tc_d_body(acc_ref, hs1_ref, dinv_ref, rbx_ref, w2a_ref, w2b_ref, b1_ref,
               x2_ref, hs2_ref):
    dinv = dinv_ref[...]
    a = acc_ref[0] + acc_ref[1] + hs1_ref[...]
    x2 = a * dinv + b1_ref[...]
    g = (jnp.dot(jnp.maximum(x2, 0.0), w2a_ref[...],
                 preferred_element_type=jnp.float32)
         + jnp.dot(jnp.maximum(rbx_ref[...], 0.0), w2b_ref[...],
                   preferred_element_type=jnp.float32))
    x2_ref[...] = x2
    hs2_ref[...] = g * dinv


def _tc_stage_d(acc1, hs1, dinv, rbx, W2a, W2b, b1):
    return pl.pallas_call(
        _tc_d_body,
        grid=(NBLK,),
        in_specs=[
            pl.BlockSpec((NC, BLK, D), lambda i: (0, i, 0)),
            pl.BlockSpec((BLK, D), lambda i: (i, 0)),
            pl.BlockSpec((BLK, 1), lambda i: (i, 0)),
            pl.BlockSpec((BLK, D), lambda i: (i, 0)),
            pl.BlockSpec((D, D), lambda i: (0, 0)),
            pl.BlockSpec((D, D), lambda i: (0, 0)),
            pl.BlockSpec((1, D), lambda i: (0, 0)),
        ],
        out_specs=[
            pl.BlockSpec((BLK, D), lambda i: (i, 0)),
            pl.BlockSpec((BLK, D), lambda i: (i, 0)),
        ],
        out_shape=[
            jax.ShapeDtypeStruct((N, D), jnp.float32),
            jax.ShapeDtypeStruct((N, D), jnp.float32),
        ],
    )(acc1, hs1, dinv, rbx, W2a, W2b, b1)


def _tc_f_body(acc2_ref, hs2_ref, dinv_ref, b2_ref, batch_ref, x2root_ref,
               out_ref, sums, cnt):
    i = pl.program_id(0)
    out2 = ((acc2_ref[0] + acc2_ref[1] + hs2_ref[...]) * dinv_ref[...]
            + b2_ref[...])
    f = jnp.maximum(out2, 0.0)
    brow = batch_ref[0]
    oh_t = (lax.broadcasted_iota(jnp.int32, (G, BLK), 0) == brow
            ).astype(jnp.float32)
    psum = jnp.dot(oh_t, f, preferred_element_type=jnp.float32)
    pcnt = jnp.dot(oh_t, jnp.ones((BLK, 1), jnp.float32),
                   preferred_element_type=jnp.float32)

    @pl.when(i == 0)
    def _():
        sums[...] = jnp.zeros_like(sums)
        cnt[...] = jnp.zeros_like(cnt)

    sums[...] += psum
    cnt[...] += pcnt

    @pl.when(i == pl.num_programs(0) - 1)
    def _():
        c = cnt[...]
        out_ref[:, :D] = sums[...] / jnp.maximum(c, 1.0)
        out_ref[:, D:] = jnp.where(c > 0.0, x2root_ref[...], 0.0)


def _tc_stage_f(acc2, hs2, dinv, b2, batch2d, x2root):
    return pl.pallas_call(
        _tc_f_body,
        grid=(NBLK,),
        in_specs=[
            pl.BlockSpec((NC, BLK, D), lambda i: (0, i, 0)),
            pl.BlockSpec((BLK, D), lambda i: (i, 0)),
            pl.BlockSpec((BLK, 1), lambda i: (i, 0)),
            pl.BlockSpec((1, D), lambda i: (0, 0)),
            pl.BlockSpec((1, 1, BLK), lambda i: (i, 0, 0)),
            pl.BlockSpec((G, D), lambda i: (0, 0)),
        ],
        out_specs=pl.BlockSpec((G, 2 * D), lambda i: (0, 0)),
        out_shape=jax.ShapeDtypeStruct((G, 2 * D), jnp.float32),
        scratch_shapes=[
            pltpu.VMEM((G, D), jnp.float32),
            pltpu.VMEM((G, 1), jnp.float32),
        ],
    )(acc2, hs2, dinv, b2, batch2d, x2root)


def kernel(x, bu_edge_index, batch, root_index, W1, b1, W2, b2):
    x = x.astype(jnp.float32)
    ei = bu_edge_index.astype(jnp.int32)
    batch_i = batch.astype(jnp.int32)
    root_i = root_index.astype(jnp.int32)

    src = ei[0]
    dst = ei[1]
    npad = EP - E
    src_p = jnp.concatenate([src, jnp.zeros((npad,), jnp.int32)])
    trash = N + (jnp.arange(npad, dtype=jnp.int32) % (NA - N))
    dst_p = jnp.concatenate([dst, trash])
    src2d = src_p.reshape(IDXROWS, CH)
    dst2d = dst_p.reshape(IDXROWS, CH)
    batch_p = jnp.concatenate([batch_i, jnp.zeros((NA - N,), jnp.int32)])

    ones_rows = jnp.ones((CH,), jnp.float32)
    zdeg = jnp.zeros((RPT,), jnp.float32)
    zrows = jnp.zeros((RPT, D), jnp.float32)

    deg, rbx_p = _sc_stage_a(dst2d, batch_p, root_i, x, ones_rows, zdeg)

    hs1, dinv = _tc_stage_b(x, W1, deg.T)

    acc1 = _sc_agg(src2d, dst2d, hs1, zrows)

    W2a = W2[:D]
    W2b = W2[D:]
    x2, hs2 = _tc_stage_d(acc1, hs1, dinv, rbx_p, W2a, W2b,
                          b1.reshape(1, D))

    acc2, x2root = _sc_agg_root(src2d, dst2d, hs2, zrows, root_i, x2)

    out = _tc_stage_f(acc2, hs2, dinv, b2.reshape(1, D),
                      batch_i.reshape(NBLK, 1, BLK), x2root)
    return out


# speedup vs baseline: 10.5922x; 1.1347x over previous
import functools

import jax
import jax.numpy as jnp
from jax import lax
from jax.experimental import pallas as pl
from jax.experimental.pallas import tpu as pltpu
from jax.experimental.pallas import tpu_sc as plsc

N = 10000
E = 320000
D = 128
G = 128

NC = 2
NS = 16
NW = NC * NS

CH = 128
ET = 10240
EP = ET * NW
NCH = ET // CH
IDXROWS = EP // CH

NA = 10240
RPT = NA // NS
DW = 16

NPT = NA // NW
RCH = 80
NRCH = NPT // RCH

BLK = 1000
NBLK = N // BLK

_MESH = plsc.VectorSubcoreMesh(core_axis_name="c", subcore_axis_name="s")


@functools.partial(
    pl.kernel,
    out_type=(
        jax.ShapeDtypeStruct((NC, NA), jnp.float32),
        jax.ShapeDtypeStruct((NA, D), jnp.float32),
    ),
    mesh=_MESH,
    scratch_types=[
        pltpu.VMEM((NCH, CH), jnp.int32),
        pltpu.VMEM((CH,), jnp.float32),
        pltpu.VMEM((G,), jnp.int32),
        pltpu.VMEM((RCH,), jnp.int32),
        pltpu.VMEM((RCH,), jnp.int32),
        pltpu.VMEM((RCH, D), jnp.float32),
        pltpu.VMEM_SHARED((NA,), jnp.float32),
        pltpu.SemaphoreType.DMA,
    ],
    compiler_params=pltpu.CompilerParams(needs_layout_passes=False),
)
def _sc_stage_a(dst2d, batch_p, root_idx, x, ones_rows, zdeg,
                deg_out, rbx_out,
                dstv, onesv, rootv, bchv, idxv, rowsv, dacc, sem):
    cid = lax.axis_index("c")
    sid = lax.axis_index("s")
    wid = cid * NS + sid

    row0 = pl.multiple_of(sid * RPT, 8)
    pltpu.sync_copy(zdeg, dacc.at[pl.ds(row0, RPT)])
    pltpu.sync_copy(ones_rows, onesv)
    pltpu.sync_copy(dst2d.at[pl.ds(pl.multiple_of(wid * NCH, 8), NCH)], dstv)
    plsc.subcore_barrier()

    def deg_body(j, carry):
        pltpu.sync_copy(onesv, dacc.at[dstv.at[j]], add=True)
        return carry
    lax.fori_loop(0, NCH, deg_body, 0)

    pltpu.sync_copy(root_idx, rootv)
    for r in range(NRCH):
        off = pl.multiple_of(wid * NPT + r * RCH, 8)
        pltpu.sync_copy(batch_p.at[pl.ds(off, RCH)], bchv)
        for gblk in range(RCH // 16):
            bvals = bchv[pl.ds(gblk * 16, 16)]
            idxv[pl.ds(gblk * 16, 16)] = plsc.load_gather(rootv, [bvals])
        pltpu.async_copy(x.at[idxv], rowsv, sem).wait()
        pltpu.sync_copy(rowsv, rbx_out.at[pl.ds(off, RCH)])

    plsc.subcore_barrier()
    pltpu.sync_copy(dacc.at[pl.ds(row0, RPT)], deg_out.at[cid, pl.ds(row0, RPT)])


NPH = 2
NCHB = NCH // NPH
NPAIR = NCHB // 2


def _make_agg(with_root):
    outs = [jax.ShapeDtypeStruct((NC, NA, D), jnp.float32)]
    scratch = [
        pltpu.VMEM((NCHB, CH), jnp.int32),
        pltpu.VMEM((NCHB, CH), jnp.int32),
        pltpu.VMEM((CH, D), jnp.float32),
        pltpu.VMEM((CH, D), jnp.float32),
        pltpu.VMEM_SHARED((NA, D), jnp.float32),
        pltpu.SemaphoreType.DMA,
        pltpu.SemaphoreType.DMA,
    ]
    if with_root:
        outs.append(jax.ShapeDtypeStruct((G, D), jnp.float32))
        scratch += [
            pltpu.VMEM((G,), jnp.int32),
        ]

    @functools.partial(
        pl.kernel,
        out_type=tuple(outs) if with_root else outs[0],
        mesh=_MESH,
        scratch_types=scratch,
    )
    def _agg(*refs):
        if with_root:
            (src2d, dst2d, table, zrows, root_idx, xroot_src,
             acc_out, root_out,
             srcv, dstv, rowsa, rowsb, acc, sema, semb, rootv) = refs
        else:
            (src2d, dst2d, table, zrows,
             acc_out,
             srcv, dstv, rowsa, rowsb, acc, sema, semb) = refs
        cid = lax.axis_index("c")
        sid = lax.axis_index("s")
        wid = cid * NS + sid

        row0 = pl.multiple_of(sid * RPT, 8)
        pltpu.sync_copy(zrows, acc.at[pl.ds(row0, RPT)])
        plsc.subcore_barrier()

        def gstart(j, buf, sem):
            pltpu.make_async_copy(table.at[srcv.at[j]], buf, sem).start()

        def gwait(j, buf, sem):
            pltpu.make_async_copy(table.at[srcv.at[j]], buf, sem).wait()

        def scat(j, buf):
            pltpu.sync_copy(buf, acc.at[dstv.at[j]], add=True)

        for ph in range(NPH):
            base = pl.multiple_of(wid * NCH + ph * NCHB, 8)
            pltpu.sync_copy(src2d.at[pl.ds(base, NCHB)], srcv)
            pltpu.sync_copy(dst2d.at[pl.ds(base, NCHB)], dstv)
            gstart(0, rowsa, sema)

            def pair_body(jj, carry):
                j0 = 2 * jj
                j1 = j0 + 1
                gstart(j1, rowsb, semb)
                gwait(j0, rowsa, sema)
                scat(j0, rowsa)

                @pl.when(jj + 1 < NPAIR)
                def _():
                    gstart(j0 + 2, rowsa, sema)

                gwait(j1, rowsb, semb)
                scat(j1, rowsb)
                return carry

            lax.fori_loop(0, NPAIR, pair_body, 0)

        if with_root:
            @pl.when(jnp.logical_and(cid == 0, sid == 0))
            def _():
                pltpu.sync_copy(root_idx, rootv)
                pltpu.async_copy(xroot_src.at[rootv], rowsa, sema).wait()
                pltpu.sync_copy(rowsa, root_out)

        plsc.subcore_barrier()
        pltpu.sync_copy(acc.at[pl.ds(row0, RPT)],
                        acc_out.at[cid, pl.ds(row0, RPT)])

    return _agg


_sc_agg = _make_agg(False)
_sc_agg_root = _make_agg(True)


def _tc_b_body(x_ref, w1_ref, deg_ref, hs1_ref, dinv_ref):
    deg = deg_ref[:, 0:1] + deg_ref[:, 1:2] + 1.0
    dinv = lax.rsqrt(deg)
    h1 = jnp.dot(x_ref[...], w1_ref[...], preferred_element_type=jnp.float32)
    hs1_ref[...] = h1 * dinv
    dinv_ref[...] = dinv


def _tc_stage_b(x, W1, deg):
    return pl.pallas_call(
        _tc_b_body,
        grid=(NBLK,),
        in_specs=[
            pl.BlockSpec((BLK, D), lambda i: (i, 0)),
            pl.BlockSpec((D, D), lambda i: (0, 0)),
            pl.BlockSpec((BLK, NC), lambda i: (i, 0)),
        ],
        out_specs=[
            pl.BlockSpec((BLK, D), lambda i: (i, 0)),
            pl.BlockSpec((BLK, 1), lambda i: (i, 0)),
        ],
        out_shape=[
            jax.ShapeDtypeStruct((N, D), jnp.float32),
            jax.ShapeDtypeStruct((N, 1), jnp.float32),
        ],
    )(x, W1, deg)


def _tc_d_body(acc_ref, hs1_ref, dinv_ref, rbx_ref, w2a_ref, w2b_ref, b1_ref,
               x2_ref, hs2_ref):
    dinv = dinv_ref[...]
    a = acc_ref[0] + acc_ref[1] + hs1_ref[...]
    x2 = a * dinv + b1_ref[...]
    g = (jnp.dot(jnp.maximum(x2, 0.0), w2a_ref[...],
                 preferred_element_type=jnp.float32)
         + jnp.dot(jnp.maximum(rbx_ref[...], 0.0), w2b_ref[...],
                   preferred_element_type=jnp.float32))
    x2_ref[...] = x2
    hs2_ref[...] = g * dinv


def _tc_stage_d(acc1, hs1, dinv, rbx, W2a, W2b, b1):
    return pl.pallas_call(
        _tc_d_body,
        grid=(NBLK,),
        in_specs=[
            pl.BlockSpec((NC, BLK, D), lambda i: (0, i, 0)),
            pl.BlockSpec((BLK, D), lambda i: (i, 0)),
            pl.BlockSpec((BLK, 1), lambda i: (i, 0)),
            pl.BlockSpec((BLK, D), lambda i: (i, 0)),
            pl.BlockSpec((D, D), lambda i: (0, 0)),
            pl.BlockSpec((D, D), lambda i: (0, 0)),
            pl.BlockSpec((1, D), lambda i: (0, 0)),
        ],
        out_specs=[
            pl.BlockSpec((BLK, D), lambda i: (i, 0)),
            pl.BlockSpec((BLK, D), lambda i: (i, 0)),
        ],
        out_shape=[
            jax.ShapeDtypeStruct((N, D), jnp.float32),
            jax.ShapeDtypeStruct((N, D), jnp.float32),
        ],
    )(acc1, hs1, dinv, rbx, W2a, W2b, b1)


def _tc_f_body(acc2_ref, hs2_ref, dinv_ref, b2_ref, batch_ref, x2root_ref,
               out_ref, sums, cnt):
    i = pl.program_id(0)
    out2 = ((acc2_ref[0] + acc2_ref[1] + hs2_ref[...]) * dinv_ref[...]
            + b2_ref[...])
    f = jnp.maximum(out2, 0.0)
    brow = batch_ref[0]
    oh_t = (lax.broadcasted_iota(jnp.int32, (G, BLK), 0) == brow
            ).astype(jnp.float32)
    psum = jnp.dot(oh_t, f, preferred_element_type=jnp.float32)
    pcnt = jnp.dot(oh_t, jnp.ones((BLK, 1), jnp.float32),
                   preferred_element_type=jnp.float32)

    @pl.when(i == 0)
    def _():
        sums[...] = jnp.zeros_like(sums)
        cnt[...] = jnp.zeros_like(cnt)

    sums[...] += psum
    cnt[...] += pcnt

    @pl.when(i == pl.num_programs(0) - 1)
    def _():
        c = cnt[...]
        out_ref[:, :D] = sums[...] / jnp.maximum(c, 1.0)
        out_ref[:, D:] = jnp.where(c > 0.0, x2root_ref[...], 0.0)


def _tc_stage_f(acc2, hs2, dinv, b2, batch2d, x2root):
    return pl.pallas_call(
        _tc_f_body,
        grid=(NBLK,),
        in_specs=[
            pl.BlockSpec((NC, BLK, D), lambda i: (0, i, 0)),
            pl.BlockSpec((BLK, D), lambda i: (i, 0)),
            pl.BlockSpec((BLK, 1), lambda i: (i, 0)),
            pl.BlockSpec((1, D), lambda i: (0, 0)),
            pl.BlockSpec((1, 1, BLK), lambda i: (i, 0, 0)),
            pl.BlockSpec((G, D), lambda i: (0, 0)),
        ],
        out_specs=pl.BlockSpec((G, 2 * D), lambda i: (0, 0)),
        out_shape=jax.ShapeDtypeStruct((G, 2 * D), jnp.float32),
        scratch_shapes=[
            pltpu.VMEM((G, D), jnp.float32),
            pltpu.VMEM((G, 1), jnp.float32),
        ],
    )(acc2, hs2, dinv, b2, batch2d, x2root)


def kernel(x, bu_edge_index, batch, root_index, W1, b1, W2, b2):
    x = x.astype(jnp.float32)
    ei = bu_edge_index.astype(jnp.int32)
    batch_i = batch.astype(jnp.int32)
    root_i = root_index.astype(jnp.int32)

    src = ei[0]
    dst = ei[1]
    npad = EP - E
    src_p = jnp.concatenate([src, jnp.zeros((npad,), jnp.int32)])
    trash = N + (jnp.arange(npad, dtype=jnp.int32) % (NA - N))
    dst_p = jnp.concatenate([dst, trash])
    src2d = src_p.reshape(IDXROWS, CH)
    dst2d = dst_p.reshape(IDXROWS, CH)
    batch_p = jnp.concatenate([batch_i, jnp.zeros((NA - N,), jnp.int32)])

    ones_rows = jnp.ones((CH,), jnp.float32)
    zdeg = jnp.zeros((RPT,), jnp.float32)
    zrows = jnp.zeros((RPT, D), jnp.float32)

    deg, rbx_p = _sc_stage_a(dst2d, batch_p, root_i, x, ones_rows, zdeg)

    hs1, dinv = _tc_stage_b(x, W1, deg.T)

    acc1 = _sc_agg(src2d, dst2d, hs1, zrows)

    W2a = W2[:D]
    W2b = W2[D:]
    x2, hs2 = _tc_stage_d(acc1, hs1, dinv, rbx_p, W2a, W2b,
                          b1.reshape(1, D))

    acc2, x2root = _sc_agg_root(src2d, dst2d, hs2, zrows, root_i, x2)

    out = _tc_stage_f(acc2, hs2, dinv, b2.reshape(1, D),
                      batch_i.reshape(NBLK, 1, BLK), x2root)
    return out


# speedup vs baseline: 21.1986x; 2.0013x over previous
import functools

import jax
import jax.numpy as jnp
from jax import lax
from jax.experimental import pallas as pl
from jax.experimental.pallas import tpu as pltpu
from jax.experimental.pallas import tpu_sc as plsc

N = 10000
E = 320000
D = 128
G = 128

NC = 2
NS = 16
NW = NC * NS

CH = 128
ET = 10240
EP = ET * NW
NCH = ET // CH
IDXROWS = EP // CH

NA = 10240
RPT = NA // NS

NH = 5120
ACCH = 5248
ART = ACCH // NS
TPT = NH // NS

CAPT = 3072
GROWS = NW * CAPT // CH
SROWS = CAPT // CH

NPT = NA // NW
RCH = 80
NRCH = NPT // RCH

BLK = 1024
NBLK = NA // BLK
HBLK = NH // BLK

_MESH = plsc.VectorSubcoreMesh(core_axis_name="c", subcore_axis_name="s")


@functools.partial(
    pl.kernel,
    out_type=(
        jax.ShapeDtypeStruct((NC, NA), jnp.float32),
        jax.ShapeDtypeStruct((NA, D), jnp.float32),
    ),
    mesh=_MESH,
    scratch_types=[
        pltpu.VMEM((NCH, CH), jnp.int32),
        pltpu.VMEM((CH,), jnp.float32),
        pltpu.VMEM((G,), jnp.int32),
        pltpu.VMEM((RCH,), jnp.int32),
        pltpu.VMEM((RCH,), jnp.int32),
        pltpu.VMEM((RCH, D), jnp.float32),
        pltpu.VMEM_SHARED((NA,), jnp.float32),
        pltpu.SemaphoreType.DMA,
    ],
    compiler_params=pltpu.CompilerParams(needs_layout_passes=False),
)
def _sc_stage_a(dst2d, batch_p, root_idx, x, ones_rows, zdeg,
                deg_out, rbx_out,
                dstv, onesv, rootv, bchv, idxv, rowsv, dacc, sem):
    cid = lax.axis_index("c")
    sid = lax.axis_index("s")
    wid = cid * NS + sid

    row0 = pl.multiple_of(sid * RPT, 8)
    pltpu.sync_copy(zdeg, dacc.at[pl.ds(row0, RPT)])
    pltpu.sync_copy(ones_rows, onesv)
    pltpu.sync_copy(dst2d.at[pl.ds(pl.multiple_of(wid * NCH, 8), NCH)], dstv)
    plsc.subcore_barrier()

    def deg_body(j, carry):
        pltpu.sync_copy(onesv, dacc.at[dstv.at[j]], add=True)
        return carry
    lax.fori_loop(0, NCH, deg_body, 0)

    pltpu.sync_copy(root_idx, rootv)
    for r in range(NRCH):
        off = pl.multiple_of(wid * NPT + r * RCH, 8)
        pltpu.sync_copy(batch_p.at[pl.ds(off, RCH)], bchv)
        for gblk in range(RCH // 16):
            bvals = bchv[pl.ds(gblk * 16, 16)]
            idxv[pl.ds(gblk * 16, 16)] = plsc.load_gather(rootv, [bvals])
        pltpu.async_copy(x.at[idxv], rowsv, sem).wait()
        pltpu.sync_copy(rowsv, rbx_out.at[pl.ds(off, RCH)])

    plsc.subcore_barrier()
    pltpu.sync_copy(dacc.at[pl.ds(row0, RPT)], deg_out.at[cid, pl.ds(row0, RPT)])


@functools.partial(
    pl.kernel,
    out_type=(
        jax.ShapeDtypeStruct((4, NW, CAPT), jnp.int32),
        jax.ShapeDtypeStruct((4, NW, CAPT), jnp.int32),
        jax.ShapeDtypeStruct((NW, 16), jnp.int32),
    ),
    mesh=_MESH,
    scratch_types=[
        pltpu.VMEM((NCH, CH), jnp.int32),
        pltpu.VMEM((NCH, CH), jnp.int32),
        pltpu.VMEM((4 * CAPT + 16,), jnp.int32),
        pltpu.VMEM((4 * CAPT + 16,), jnp.int32),
        pltpu.VMEM((16,), jnp.int32),
    ],
    compiler_params=pltpu.CompilerParams(needs_layout_passes=False),
)
def _sc_partition(src2d, dst2d,
                  gsrc_out, gdst_out, cnt_out,
                  srcv, dstv, gs, gd, cntv):
    cid = lax.axis_index("c")
    sid = lax.axis_index("s")
    wid = cid * NS + sid

    base = pl.multiple_of(wid * NCH, 8)
    pltpu.sync_copy(src2d.at[pl.ds(base, NCH)], srcv)
    pltpu.sync_copy(dst2d.at[pl.ds(base, NCH)], dstv)

    lanes = lax.iota(jnp.int32, 16)
    padsrc = jnp.zeros((16,), jnp.int32)
    paddst = NH + 8 * lanes
    def fill_body(i, carry):
        gs[pl.ds(i * 16, 16)] = padsrc
        gd[pl.ds(i * 16, 16)] = paddst
        return carry
    lax.fori_loop(0, (4 * CAPT) // 16, fill_body, 0)

    def chunk_body(j, offs):
        for k in range(CH // 16):
            sv = srcv[j, pl.ds(k * 16, 16)]
            dv = dstv[j, pl.ds(k * 16, 16)]
            ps = jnp.where(sv >= NH, 1, 0)
            qd = jnp.where(dv >= NH, 1, 0)
            srcl = sv - ps * NH
            dstl = dv - qd * NH
            grp = ps + 2 * qd
            new_offs = []
            for g in range(4):
                m = grp == g
                off = offs[g]
                plsc.store_compressed(gd.at[pl.ds(g * CAPT + off, 16)],
                                      dstl, mask=m)
                plsc.store_compressed(gs.at[pl.ds(g * CAPT + off, 16)],
                                      srcl, mask=m)
                pc = jnp.max(plsc.all_reduce_population_count(m))
                new_offs.append(off + pc)
            offs = tuple(new_offs)
        return offs

    offs = lax.fori_loop(0, NCH, chunk_body,
                         (jnp.int32(0), jnp.int32(0),
                          jnp.int32(0), jnp.int32(0)))

    for g in range(4):
        pltpu.sync_copy(gs.at[pl.ds(g * CAPT, CAPT)], gsrc_out.at[g, wid])
        pltpu.sync_copy(gd.at[pl.ds(g * CAPT, CAPT)], gdst_out.at[g, wid])
    lanes4 = lax.iota(jnp.int32, 16)
    cvec = jnp.where(lanes4 == 0, offs[0],
                     jnp.where(lanes4 == 1, offs[1],
                               jnp.where(lanes4 == 2, offs[2],
                                         jnp.where(lanes4 == 3, offs[3], 0))))
    cntv[...] = cvec
    pltpu.sync_copy(cntv, cnt_out.at[wid])


def _make_agg(with_root):
    outs = [jax.ShapeDtypeStruct((NC, ACCH, D), jnp.float32)]
    scratch = [
        pltpu.VMEM((SROWS, CH), jnp.int32),
        pltpu.VMEM((SROWS, CH), jnp.int32),
        pltpu.VMEM((CH, D), jnp.float32),
        pltpu.VMEM((CH, D), jnp.float32),
        pltpu.VMEM((NW, 16), jnp.int32),
        pltpu.VMEM_SHARED((ACCH, D), jnp.float32),
        pltpu.VMEM_SHARED((NH, D), jnp.float32),
        pltpu.SemaphoreType.DMA,
        pltpu.SemaphoreType.DMA,
    ]
    if with_root:
        outs.append(jax.ShapeDtypeStruct((G, D), jnp.float32))
        scratch += [pltpu.VMEM((G,), jnp.int32)]

    @functools.partial(
        pl.kernel,
        out_type=tuple(outs) if with_root else outs[0],
        mesh=_MESH,
        scratch_types=scratch,
        compiler_params=pltpu.CompilerParams(needs_layout_passes=False),
    )
    def _agg(*refs):
        if with_root:
            (gsrc, gdst, cnts, table, zrows, root_idx, xroot_src,
             acc_out, root_out,
             srcv, dstv, rowsa, rowsb, cntv, acc, tab, sema, semb,
             rootv) = refs
        else:
            (gsrc, gdst, cnts, table, zrows,
             acc_out,
             srcv, dstv, rowsa, rowsb, cntv, acc, tab, sema, semb) = refs
        cid = lax.axis_index("c")
        sid = lax.axis_index("s")

        row0 = pl.multiple_of(sid * ART, 8)
        pltpu.sync_copy(zrows, acc.at[pl.ds(row0, ART)])
        pltpu.sync_copy(cnts, cntv)

        def gstart(j, buf, sem):
            pltpu.make_async_copy(tab.at[srcv.at[j]], buf, sem).start()

        def gwait(j, buf, sem):
            pltpu.make_async_copy(tab.at[srcv.at[j]], buf, sem).wait()

        def scat(j, buf):
            pltpu.sync_copy(buf, acc.at[dstv.at[j]], add=True)

        for ps in range(2):
            p = cid if ps == 0 else 1 - cid
            g = 3 * cid if ps == 0 else 1 + cid
            plsc.subcore_barrier()
            toff = p * NH + sid * TPT
            pltpu.sync_copy(table.at[pl.ds(toff, TPT)],
                            tab.at[pl.ds(pl.multiple_of(sid * TPT, 8), TPT)])
            plsc.subcore_barrier()

            for sl in range(2):
                w2 = 2 * sid + sl
                cv = cntv[w2, pl.ds(0, 16)]
                cnt = jnp.sum(jnp.where(lax.iota(jnp.int32, 16) == g, cv, 0))
                npair = (cnt + 2 * CH - 1) // (2 * CH)
                srow = pl.multiple_of(w2 * SROWS, 8)
                pltpu.sync_copy(gsrc.at[ps, cid, pl.ds(srow, SROWS)], srcv)
                pltpu.sync_copy(gdst.at[ps, cid, pl.ds(srow, SROWS)], dstv)

                @pl.when(cnt > 0)
                def _():
                    b0 = 0
                    gstart(b0, rowsa, sema)

                    def pair_body(jj, carry):
                        j0 = b0 + 2 * jj
                        j1 = j0 + 1
                        gstart(j1, rowsb, semb)
                        gwait(j0, rowsa, sema)
                        scat(j0, rowsa)

                        @pl.when(jj + 1 < npair)
                        def _():
                            gstart(j0 + 2, rowsa, sema)

                        gwait(j1, rowsb, semb)
                        scat(j1, rowsb)
                        return carry

                    lax.fori_loop(0, npair, pair_body, 0)

        if with_root:
            @pl.when(jnp.logical_and(cid == 0, sid == 0))
            def _():
                pltpu.sync_copy(root_idx, rootv)
                pltpu.async_copy(xroot_src.at[rootv], rowsa, sema).wait()
                pltpu.sync_copy(rowsa, root_out)

        plsc.subcore_barrier()
        pltpu.sync_copy(acc.at[pl.ds(row0, ART)],
                        acc_out.at[cid, pl.ds(row0, ART)])

    return _agg


_sc_agg = _make_agg(False)
_sc_agg_root = _make_agg(True)


def _tc_b_body(x_ref, w1_ref, deg_ref, hs1_ref, dinv_ref):
    deg = deg_ref[:, 0:1] + deg_ref[:, 1:2] + 1.0
    dinv = lax.rsqrt(deg)
    h1 = jnp.dot(x_ref[...], w1_ref[...], preferred_element_type=jnp.float32)
    hs1_ref[...] = h1 * dinv
    dinv_ref[...] = dinv


def _tc_stage_b(x, W1, deg):
    return pl.pallas_call(
        _tc_b_body,
        grid=(NBLK,),
        in_specs=[
            pl.BlockSpec((BLK, D), lambda i: (i, 0)),
            pl.BlockSpec((D, D), lambda i: (0, 0)),
            pl.BlockSpec((BLK, NC), lambda i: (i, 0)),
        ],
        out_specs=[
            pl.BlockSpec((BLK, D), lambda i: (i, 0)),
            pl.BlockSpec((BLK, 1), lambda i: (i, 0)),
        ],
        out_shape=[
            jax.ShapeDtypeStruct((NA, D), jnp.float32),
            jax.ShapeDtypeStruct((NA, 1), jnp.float32),
        ],
    )(x, W1, deg)


def _acc_spec():
    return pl.BlockSpec((1, BLK, D), lambda i: (i // HBLK, i % HBLK, 0))


def _tc_d_body(acc_ref, hs1_ref, dinv_ref, rbx_ref, w2a_ref, w2b_ref, b1_ref,
               x2_ref, hs2_ref):
    dinv = dinv_ref[...]
    x2 = (acc_ref[0] + hs1_ref[...]) * dinv + b1_ref[...]
    g = (jnp.dot(jnp.maximum(x2, 0.0), w2a_ref[...],
                 preferred_element_type=jnp.float32)
         + jnp.dot(jnp.maximum(rbx_ref[...], 0.0), w2b_ref[...],
                   preferred_element_type=jnp.float32))
    x2_ref[...] = x2
    hs2_ref[...] = g * dinv


def _tc_stage_d(acc1, hs1, dinv, rbx, W2a, W2b, b1):
    return pl.pallas_call(
        _tc_d_body,
        grid=(NBLK,),
        in_specs=[
            _acc_spec(),
            pl.BlockSpec((BLK, D), lambda i: (i, 0)),
            pl.BlockSpec((BLK, 1), lambda i: (i, 0)),
            pl.BlockSpec((BLK, D), lambda i: (i, 0)),
            pl.BlockSpec((D, D), lambda i: (0, 0)),
            pl.BlockSpec((D, D), lambda i: (0, 0)),
            pl.BlockSpec((1, D), lambda i: (0, 0)),
        ],
        out_specs=[
            pl.BlockSpec((BLK, D), lambda i: (i, 0)),
            pl.BlockSpec((BLK, D), lambda i: (i, 0)),
        ],
        out_shape=[
            jax.ShapeDtypeStruct((NA, D), jnp.float32),
            jax.ShapeDtypeStruct((NA, D), jnp.float32),
        ],
    )(acc1, hs1, dinv, rbx, W2a, W2b, b1)


def _tc_f_body(acc2_ref, hs2_ref, dinv_ref, b2_ref, batch_ref, x2root_ref,
               out_ref, sums, cnt):
    i = pl.program_id(0)
    out2 = (acc2_ref[0] + hs2_ref[...]) * dinv_ref[...] + b2_ref[...]
    f = jnp.maximum(out2, 0.0)
    brow = batch_ref[0]
    oh_t = (lax.broadcasted_iota(jnp.int32, (G, BLK), 0) == brow
            ).astype(jnp.float32)
    psum = jnp.dot(oh_t, f, preferred_element_type=jnp.float32)
    pcnt = jnp.dot(oh_t, jnp.ones((BLK, 1), jnp.float32),
                   preferred_element_type=jnp.float32)

    @pl.when(i == 0)
    def _():
        sums[...] = jnp.zeros_like(sums)
        cnt[...] = jnp.zeros_like(cnt)

    sums[...] += psum
    cnt[...] += pcnt

    @pl.when(i == pl.num_programs(0) - 1)
    def _():
        c = cnt[...]
        out_ref[:, :D] = sums[...] / jnp.maximum(c, 1.0)
        out_ref[:, D:] = jnp.where(c > 0.0, x2root_ref[...], 0.0)


def _tc_stage_f(acc2, hs2, dinv, b2, batch3d, x2root):
    return pl.pallas_call(
        _tc_f_body,
        grid=(NBLK,),
        in_specs=[
            _acc_spec(),
            pl.BlockSpec((BLK, D), lambda i: (i, 0)),
            pl.BlockSpec((BLK, 1), lambda i: (i, 0)),
            pl.BlockSpec((1, D), lambda i: (0, 0)),
            pl.BlockSpec((1, 1, BLK), lambda i: (i, 0, 0)),
            pl.BlockSpec((G, D), lambda i: (0, 0)),
        ],
        out_specs=pl.BlockSpec((G, 2 * D), lambda i: (0, 0)),
        out_shape=jax.ShapeDtypeStruct((G, 2 * D), jnp.float32),
        scratch_shapes=[
            pltpu.VMEM((G, D), jnp.float32),
            pltpu.VMEM((G, 1), jnp.float32),
        ],
    )(acc2, hs2, dinv, b2, batch3d, x2root)


def kernel(x, bu_edge_index, batch, root_index, W1, b1, W2, b2):
    x = x.astype(jnp.float32)
    ei = bu_edge_index.astype(jnp.int32)
    batch_i = batch.astype(jnp.int32)
    root_i = root_index.astype(jnp.int32)

    src = ei[0]
    dst = ei[1]
    npad = EP - E
    ppt = npad // NW
    src_p = jnp.concatenate(
        [src.reshape(NW, E // NW),
         jnp.zeros((NW, ppt), jnp.int32)], axis=1).reshape(-1)
    trash = N + (jnp.arange(ppt, dtype=jnp.int32) % (NA - N))
    dst_p = jnp.concatenate(
        [dst.reshape(NW, E // NW),
         jnp.broadcast_to(trash, (NW, ppt))], axis=1).reshape(-1)
    src2d = src_p.reshape(IDXROWS, CH)
    dst2d = dst_p.reshape(IDXROWS, CH)
    batch_p = jnp.concatenate([batch_i, jnp.zeros((NA - N,), jnp.int32)])
    batch_f = jnp.concatenate(
        [batch_i, jnp.full((NA - N,), -1, jnp.int32)])
    x_p = jnp.concatenate([x, jnp.zeros((NA - N, D), jnp.float32)])

    ones_deg = jnp.ones((CH,), jnp.float32)
    zdeg = jnp.zeros((RPT,), jnp.float32)
    zrows = jnp.zeros((ART, D), jnp.float32)

    deg, rbx_p = _sc_stage_a(dst2d, batch_p, root_i, x, ones_deg, zdeg)

    gsrc4, gdst4, cnts = _sc_partition(src2d, dst2d)
    order = jnp.array([0, 3, 1, 2], jnp.int32)
    gsrc = gsrc4[order].reshape(2, NC, GROWS, CH)
    gdst = gdst4[order].reshape(2, NC, GROWS, CH)

    hs1, dinv = _tc_stage_b(x_p, W1, deg.T)

    acc1 = _sc_agg(gsrc, gdst, cnts, hs1, zrows)

    W2a = W2[:D]
    W2b = W2[D:]
    x2, hs2 = _tc_stage_d(acc1, hs1, dinv, rbx_p, W2a, W2b,
                          b1.reshape(1, D))

    acc2, x2root = _sc_agg_root(gsrc, gdst, cnts, hs2, zrows, root_i, x2)

    out = _tc_stage_f(acc2, hs2, dinv, b2.reshape(1, D),
                      batch_f.reshape(NBLK, 1, BLK), x2root)
    return out


# speedup vs baseline: 21.5017x; 1.0143x over previous
import functools

import jax
import jax.numpy as jnp
from jax import lax
from jax.experimental import pallas as pl
from jax.experimental.pallas import tpu as pltpu
from jax.experimental.pallas import tpu_sc as plsc

N = 10000
E = 320000
D = 128
G = 128

NC = 2
NS = 16
NW = NC * NS

CH = 128
ET = 10240
EP = ET * NW
NCH = ET // CH
IDXROWS = EP // CH

NA = 10240
RPT = NA // NS

NH = 5120
ACCH = 5248
ART = ACCH // NS
TPT = NH // NS

CAPT = 3072
GROWS = NW * CAPT // CH
SROWS = CAPT // CH

NPT = NA // NW
RCH = 80
NRCH = NPT // RCH

BLK = 1024
NBLK = NA // BLK
HBLK = NH // BLK

_MESH = plsc.VectorSubcoreMesh(core_axis_name="c", subcore_axis_name="s")


@functools.partial(
    pl.kernel,
    out_type=(
        jax.ShapeDtypeStruct((NC, NA), jnp.float32),
        jax.ShapeDtypeStruct((NA, D), jnp.float32),
        jax.ShapeDtypeStruct((4, NW, CAPT), jnp.int32),
        jax.ShapeDtypeStruct((4, NW, CAPT), jnp.int32),
        jax.ShapeDtypeStruct((NW, 16), jnp.int32),
    ),
    mesh=_MESH,
    scratch_types=[
        pltpu.VMEM((NCH, CH), jnp.int32),
        pltpu.VMEM((NCH, CH), jnp.int32),
        pltpu.VMEM((4 * CAPT + 16,), jnp.int32),
        pltpu.VMEM((4 * CAPT + 16,), jnp.int32),
        pltpu.VMEM((16,), jnp.int32),
        pltpu.VMEM((CH,), jnp.float32),
        pltpu.VMEM((G,), jnp.int32),
        pltpu.VMEM((RCH,), jnp.int32),
        pltpu.VMEM((RCH,), jnp.int32),
        pltpu.VMEM((RCH, D), jnp.float32),
        pltpu.VMEM_SHARED((NA,), jnp.float32),
        pltpu.SemaphoreType.DMA,
        pltpu.SemaphoreType.DMA,
    ],
    compiler_params=pltpu.CompilerParams(needs_layout_passes=False),
)
def _sc_prep(src2d, dst2d, batch_p, root_idx, x, ones_rows, zdeg,
             deg_out, rbx_out, gsrc_out, gdst_out, cnt_out,
             srcv, dstv, gs, gd, cntv, onesv, rootv, bchv, idxv, rowsv,
             dacc, sem, semd):
    cid = lax.axis_index("c")
    sid = lax.axis_index("s")
    wid = cid * NS + sid

    base = pl.multiple_of(wid * NCH, 8)
    pltpu.sync_copy(src2d.at[pl.ds(base, NCH)], srcv)
    pltpu.sync_copy(dst2d.at[pl.ds(base, NCH)], dstv)
    row0 = pl.multiple_of(sid * RPT, 8)
    pltpu.sync_copy(zdeg, dacc.at[pl.ds(row0, RPT)])
    pltpu.sync_copy(ones_rows, onesv)
    plsc.subcore_barrier()

    lanes = lax.iota(jnp.int32, 16)
    padsrc = jnp.zeros((16,), jnp.int32)
    paddst = NH + 8 * lanes
    def fill_body(i, carry):
        gs[pl.ds(i * 16, 16)] = padsrc
        gd[pl.ds(i * 16, 16)] = paddst
        return carry
    lax.fori_loop(0, (4 * CAPT) // 16, fill_body, 0)

    def chunk_body(j, offs):
        pltpu.async_copy(onesv, dacc.at[dstv.at[j]], semd, add=True)
        for k in range(CH // 16):
            sv = srcv[j, pl.ds(k * 16, 16)]
            dv = dstv[j, pl.ds(k * 16, 16)]
            ps = jnp.where(sv >= NH, 1, 0)
            qd = jnp.where(dv >= NH, 1, 0)
            srcl = sv - ps * NH
            dstl = dv - qd * NH
            grp = ps + 2 * qd
            new_offs = []
            for g in range(4):
                m = grp == g
                off = offs[g]
                plsc.store_compressed(gd.at[pl.ds(g * CAPT + off, 16)],
                                      dstl, mask=m)
                plsc.store_compressed(gs.at[pl.ds(g * CAPT + off, 16)],
                                      srcl, mask=m)
                pc = jnp.max(plsc.all_reduce_population_count(m))
                new_offs.append(off + pc)
            offs = tuple(new_offs)
        return offs

    offs = lax.fori_loop(0, NCH, chunk_body,
                         (jnp.int32(0), jnp.int32(0),
                          jnp.int32(0), jnp.int32(0)))

    def drain_body(j, carry):
        pltpu.make_async_copy(onesv, dacc.at[dstv.at[0]], semd).wait()
        return carry
    lax.fori_loop(0, NCH, drain_body, 0)

    for g in range(4):
        pltpu.sync_copy(gs.at[pl.ds(g * CAPT, CAPT)], gsrc_out.at[g, wid])
        pltpu.sync_copy(gd.at[pl.ds(g * CAPT, CAPT)], gdst_out.at[g, wid])
    lanes4 = lax.iota(jnp.int32, 16)
    cvec = jnp.where(lanes4 == 0, offs[0],
                     jnp.where(lanes4 == 1, offs[1],
                               jnp.where(lanes4 == 2, offs[2],
                                         jnp.where(lanes4 == 3, offs[3], 0))))
    cntv[...] = cvec
    pltpu.sync_copy(cntv, cnt_out.at[wid])

    pltpu.sync_copy(root_idx, rootv)
    for r in range(NRCH):
        off = pl.multiple_of(wid * NPT + r * RCH, 8)
        pltpu.sync_copy(batch_p.at[pl.ds(off, RCH)], bchv)
        for gblk in range(RCH // 16):
            bvals = bchv[pl.ds(gblk * 16, 16)]
            idxv[pl.ds(gblk * 16, 16)] = plsc.load_gather(rootv, [bvals])
        pltpu.async_copy(x.at[idxv], rowsv, sem).wait()
        pltpu.sync_copy(rowsv, rbx_out.at[pl.ds(off, RCH)])

    plsc.subcore_barrier()
    pltpu.sync_copy(dacc.at[pl.ds(row0, RPT)], deg_out.at[cid, pl.ds(row0, RPT)])


def _make_agg(with_root):
    outs = [jax.ShapeDtypeStruct((NC, ACCH, D), jnp.float32)]
    scratch = [
        pltpu.VMEM((SROWS, CH), jnp.int32),
        pltpu.VMEM((SROWS, CH), jnp.int32),
        pltpu.VMEM((CH, D), jnp.float32),
        pltpu.VMEM((CH, D), jnp.float32),
        pltpu.VMEM((NW, 16), jnp.int32),
        pltpu.VMEM_SHARED((ACCH, D), jnp.float32),
        pltpu.VMEM_SHARED((NH, D), jnp.float32),
        pltpu.SemaphoreType.DMA,
        pltpu.SemaphoreType.DMA,
    ]
    if with_root:
        outs.append(jax.ShapeDtypeStruct((G, D), jnp.float32))
        scratch += [pltpu.VMEM((G,), jnp.int32)]

    @functools.partial(
        pl.kernel,
        out_type=tuple(outs) if with_root else outs[0],
        mesh=_MESH,
        scratch_types=scratch,
        compiler_params=pltpu.CompilerParams(needs_layout_passes=False),
    )
    def _agg(*refs):
        if with_root:
            (gsrc, gdst, cnts, table, zrows, root_idx, xroot_src,
             acc_out, root_out,
             srcv, dstv, rowsa, rowsb, cntv, acc, tab, sema, semb,
             rootv) = refs
        else:
            (gsrc, gdst, cnts, table, zrows,
             acc_out,
             srcv, dstv, rowsa, rowsb, cntv, acc, tab, sema, semb) = refs
        cid = lax.axis_index("c")
        sid = lax.axis_index("s")

        row0 = pl.multiple_of(sid * ART, 8)
        pltpu.sync_copy(zrows, acc.at[pl.ds(row0, ART)])
        pltpu.sync_copy(cnts, cntv)

        def gstart(j, buf, sem):
            pltpu.make_async_copy(tab.at[srcv.at[j]], buf, sem).start()

        def gwait(j, buf, sem):
            pltpu.make_async_copy(tab.at[srcv.at[j]], buf, sem).wait()

        def scat(j, buf):
            pltpu.sync_copy(buf, acc.at[dstv.at[j]], add=True)

        for ps in range(2):
            p = cid if ps == 0 else 1 - cid
            g = 3 * cid if ps == 0 else 1 + cid
            plsc.subcore_barrier()
            toff = p * NH + sid * TPT
            pltpu.sync_copy(table.at[pl.ds(toff, TPT)],
                            tab.at[pl.ds(pl.multiple_of(sid * TPT, 8), TPT)])
            plsc.subcore_barrier()

            for sl in range(2):
                w2 = 2 * sid + sl
                cv = cntv[w2, pl.ds(0, 16)]
                cnt = jnp.sum(jnp.where(lax.iota(jnp.int32, 16) == g, cv, 0))
                npair = (cnt + 2 * CH - 1) // (2 * CH)
                srow = pl.multiple_of(w2 * SROWS, 8)
                pltpu.sync_copy(gsrc.at[ps, cid, pl.ds(srow, SROWS)], srcv)
                pltpu.sync_copy(gdst.at[ps, cid, pl.ds(srow, SROWS)], dstv)

                @pl.when(cnt > 0)
                def _():
                    b0 = 0
                    gstart(b0, rowsa, sema)

                    def pair_body(jj, carry):
                        j0 = b0 + 2 * jj
                        j1 = j0 + 1
                        gstart(j1, rowsb, semb)
                        gwait(j0, rowsa, sema)
                        scat(j0, rowsa)

                        @pl.when(jj + 1 < npair)
                        def _():
                            gstart(j0 + 2, rowsa, sema)

                        gwait(j1, rowsb, semb)
                        scat(j1, rowsb)
                        return carry

                    lax.fori_loop(0, npair, pair_body, 0)

        if with_root:
            @pl.when(jnp.logical_and(cid == 0, sid == 0))
            def _():
                pltpu.sync_copy(root_idx, rootv)
                pltpu.async_copy(xroot_src.at[rootv], rowsa, sema).wait()
                pltpu.sync_copy(rowsa, root_out)

        plsc.subcore_barrier()
        pltpu.sync_copy(acc.at[pl.ds(row0, ART)],
                        acc_out.at[cid, pl.ds(row0, ART)])

    return _agg


_sc_agg = _make_agg(False)
_sc_agg_root = _make_agg(True)


def _tc_b_body(x_ref, w1_ref, deg_ref, hs1_ref, dinv_ref):
    deg = deg_ref[:, 0:1] + deg_ref[:, 1:2] + 1.0
    dinv = lax.rsqrt(deg)
    h1 = jnp.dot(x_ref[...], w1_ref[...], preferred_element_type=jnp.float32)
    hs1_ref[...] = h1 * dinv
    dinv_ref[...] = dinv


def _tc_stage_b(x, W1, deg):
    return pl.pallas_call(
        _tc_b_body,
        grid=(NBLK,),
        in_specs=[
            pl.BlockSpec((BLK, D), lambda i: (i, 0)),
            pl.BlockSpec((D, D), lambda i: (0, 0)),
            pl.BlockSpec((BLK, NC), lambda i: (i, 0)),
        ],
        out_specs=[
            pl.BlockSpec((BLK, D), lambda i: (i, 0)),
            pl.BlockSpec((BLK, 1), lambda i: (i, 0)),
        ],
        out_shape=[
            jax.ShapeDtypeStruct((NA, D), jnp.float32),
            jax.ShapeDtypeStruct((NA, 1), jnp.float32),
        ],
    )(x, W1, deg)


def _acc_spec():
    return pl.BlockSpec((1, BLK, D), lambda i: (i // HBLK, i % HBLK, 0))


def _tc_d_body(acc_ref, hs1_ref, dinv_ref, rbx_ref, w2a_ref, w2b_ref, b1_ref,
               x2_ref, hs2_ref):
    dinv = dinv_ref[...]
    x2 = (acc_ref[0] + hs1_ref[...]) * dinv + b1_ref[...]
    g = (jnp.dot(jnp.maximum(x2, 0.0), w2a_ref[...],
                 preferred_element_type=jnp.float32)
         + jnp.dot(jnp.maximum(rbx_ref[...], 0.0), w2b_ref[...],
                   preferred_element_type=jnp.float32))
    x2_ref[...] = x2
    hs2_ref[...] = g * dinv


def _tc_stage_d(acc1, hs1, dinv, rbx, W2a, W2b, b1):
    return pl.pallas_call(
        _tc_d_body,
        grid=(NBLK,),
        in_specs=[
            _acc_spec(),
            pl.BlockSpec((BLK, D), lambda i: (i, 0)),
            pl.BlockSpec((BLK, 1), lambda i: (i, 0)),
            pl.BlockSpec((BLK, D), lambda i: (i, 0)),
            pl.BlockSpec((D, D), lambda i: (0, 0)),
            pl.BlockSpec((D, D), lambda i: (0, 0)),
            pl.BlockSpec((1, D), lambda i: (0, 0)),
        ],
        out_specs=[
            pl.BlockSpec((BLK, D), lambda i: (i, 0)),
            pl.BlockSpec((BLK, D), lambda i: (i, 0)),
        ],
        out_shape=[
            jax.ShapeDtypeStruct((NA, D), jnp.float32),
            jax.ShapeDtypeStruct((NA, D), jnp.float32),
        ],
    )(acc1, hs1, dinv, rbx, W2a, W2b, b1)


def _tc_f_body(acc2_ref, hs2_ref, dinv_ref, b2_ref, batch_ref, x2root_ref,
               out_ref, sums, cnt):
    i = pl.program_id(0)
    out2 = (acc2_ref[0] + hs2_ref[...]) * dinv_ref[...] + b2_ref[...]
    f = jnp.maximum(out2, 0.0)
    brow = batch_ref[0]
    oh_t = (lax.broadcasted_iota(jnp.int32, (G, BLK), 0) == brow
            ).astype(jnp.float32)
    psum = jnp.dot(oh_t, f, preferred_element_type=jnp.float32)
    pcnt = jnp.dot(oh_t, jnp.ones((BLK, 1), jnp.float32),
                   preferred_element_type=jnp.float32)

    @pl.when(i == 0)
    def _():
        sums[...] = jnp.zeros_like(sums)
        cnt[...] = jnp.zeros_like(cnt)

    sums[...] += psum
    cnt[...] += pcnt

    @pl.when(i == pl.num_programs(0) - 1)
    def _():
        c = cnt[...]
        out_ref[:, :D] = sums[...] / jnp.maximum(c, 1.0)
        out_ref[:, D:] = jnp.where(c > 0.0, x2root_ref[...], 0.0)


def _tc_stage_f(acc2, hs2, dinv, b2, batch3d, x2root):
    return pl.pallas_call(
        _tc_f_body,
        grid=(NBLK,),
        in_specs=[
            _acc_spec(),
            pl.BlockSpec((BLK, D), lambda i: (i, 0)),
            pl.BlockSpec((BLK, 1), lambda i: (i, 0)),
            pl.BlockSpec((1, D), lambda i: (0, 0)),
            pl.BlockSpec((1, 1, BLK), lambda i: (i, 0, 0)),
            pl.BlockSpec((G, D), lambda i: (0, 0)),
        ],
        out_specs=pl.BlockSpec((G, 2 * D), lambda i: (0, 0)),
        out_shape=jax.ShapeDtypeStruct((G, 2 * D), jnp.float32),
        scratch_shapes=[
            pltpu.VMEM((G, D), jnp.float32),
            pltpu.VMEM((G, 1), jnp.float32),
        ],
    )(acc2, hs2, dinv, b2, batch3d, x2root)


def kernel(x, bu_edge_index, batch, root_index, W1, b1, W2, b2):
    x = x.astype(jnp.float32)
    ei = bu_edge_index.astype(jnp.int32)
    batch_i = batch.astype(jnp.int32)
    root_i = root_index.astype(jnp.int32)

    src = ei[0]
    dst = ei[1]
    npad = EP - E
    ppt = npad // NW
    src_p = jnp.concatenate(
        [src.reshape(NW, E // NW),
         jnp.zeros((NW, ppt), jnp.int32)], axis=1).reshape(-1)
    trash = N + (jnp.arange(ppt, dtype=jnp.int32) % (NA - N))
    dst_p = jnp.concatenate(
        [dst.reshape(NW, E // NW),
         jnp.broadcast_to(trash, (NW, ppt))], axis=1).reshape(-1)
    src2d = src_p.reshape(IDXROWS, CH)
    dst2d = dst_p.reshape(IDXROWS, CH)
    batch_p = jnp.concatenate([batch_i, jnp.zeros((NA - N,), jnp.int32)])
    batch_f = jnp.concatenate(
        [batch_i, jnp.full((NA - N,), -1, jnp.int32)])
    x_p = jnp.concatenate([x, jnp.zeros((NA - N, D), jnp.float32)])

    ones_deg = jnp.ones((CH,), jnp.float32)
    zdeg = jnp.zeros((RPT,), jnp.float32)
    zrows = jnp.zeros((ART, D), jnp.float32)

    deg, rbx_p, gsrc4, gdst4, cnts = _sc_prep(
        src2d, dst2d, batch_p, root_i, x, ones_deg, zdeg)
    order = jnp.array([0, 3, 1, 2], jnp.int32)
    gsrc = gsrc4[order].reshape(2, NC, GROWS, CH)
    gdst = gdst4[order].reshape(2, NC, GROWS, CH)

    hs1, dinv = _tc_stage_b(x_p, W1, deg.T)

    acc1 = _sc_agg(gsrc, gdst, cnts, hs1, zrows)

    W2a = W2[:D]
    W2b = W2[D:]
    x2, hs2 = _tc_stage_d(acc1, hs1, dinv, rbx_p, W2a, W2b,
                          b1.reshape(1, D))

    acc2, x2root = _sc_agg_root(gsrc, gdst, cnts, hs2, zrows, root_i, x2)

    out = _tc_stage_f(acc2, hs2, dinv, b2.reshape(1, D),
                      batch_f.reshape(NBLK, 1, BLK), x2root)
    return out
